# baseline (speedup 1.0000x reference)
import functools

import jax
import jax.numpy as jnp
from jax import lax
from jax.experimental import pallas as pl
from jax.experimental.pallas import tpu as pltpu
from jax.experimental.pallas import tpu_sc as plsc

B = 128
T = 2048
V = 29
NUM_CORES = 2
NUM_SUBCORES = 16
NW = NUM_CORES * NUM_SUBCORES
ROWS_PER_W = B // NW
ROW_W = T * V

_mesh = plsc.VectorSubcoreMesh(
    core_axis_name="c", subcore_axis_name="s",
    num_cores=NUM_CORES, num_subcores=NUM_SUBCORES,
)


@functools.partial(
    pl.kernel,
    out_type=[
        jax.ShapeDtypeStruct((B * T,), jnp.int32),
        jax.ShapeDtypeStruct((B * T,), jnp.int32),
        jax.ShapeDtypeStruct((B * T,), jnp.float32),
    ],
    mesh=_mesh,
    compiler_params=pltpu.CompilerParams(needs_layout_passes=False),
    scratch_types=[
        pltpu.VMEM((ROW_W,), jnp.float32),
        pltpu.VMEM((T,), jnp.int32),
        pltpu.VMEM((T,), jnp.int32),
        pltpu.VMEM((T,), jnp.float32),
    ],
)
def _ctc_sc(lp_hbm, idx_hbm, val_hbm, mp_hbm, inbuf, idxrow, valrow, mprow):
    wid = lax.axis_index("s") * NUM_CORES + lax.axis_index("c")
    iota16 = lax.iota(jnp.int32, 16)

    def do_row(b):
        pltpu.sync_copy(lp_hbm.at[pl.ds(pl.multiple_of(b * ROW_W, 8), ROW_W)], inbuf)

        def group(g, carry):
            t = g * 16 + iota16
            base = t * V
            cmax = plsc.load_gather(inbuf, [base])
            cidx = jnp.zeros((16,), jnp.int32)
            for v in range(1, V):
                x = plsc.load_gather(inbuf, [base + v])
                gt = x > cmax
                cidx = jnp.where(gt, v, cidx)
                cmax = jnp.where(gt, x, cmax)
            plsc.store_scatter(idxrow, [t], cidx)
            prev = plsc.load_gather(idxrow, [jnp.maximum(t - 1, 0)])
            valid = (cidx != 0) & ((cidx != prev) | (t == 0))
            plsc.store_scatter(valrow, [t], valid.astype(jnp.int32))
            plsc.store_scatter(mprow, [t], jnp.exp(cmax))
            return carry

        lax.fori_loop(0, T // 16, group, 0)

        off = pl.multiple_of(b * T, 8)
        pltpu.sync_copy(idxrow, idx_hbm.at[pl.ds(off, T)])
        pltpu.sync_copy(valrow, val_hbm.at[pl.ds(off, T)])
        pltpu.sync_copy(mprow, mp_hbm.at[pl.ds(off, T)])

    for r in range(ROWS_PER_W):
        do_row(wid * ROWS_PER_W + r)


def kernel(log_probs):
    lp = log_probs.reshape(-1)
    idx, val, mp = _ctc_sc(lp)
    return (
        idx.reshape(B, T),
        val.reshape(B, T).astype(bool),
        mp.reshape(B, T),
    )


# speedup vs baseline: 1.0425x; 1.0425x over previous
import functools

import jax
import jax.numpy as jnp
from jax import lax
from jax.experimental import pallas as pl
from jax.experimental.pallas import tpu as pltpu
from jax.experimental.pallas import tpu_sc as plsc

B = 128
T = 2048
V = 29
NUM_CORES = 2
NUM_SUBCORES = 16
NW = NUM_CORES * NUM_SUBCORES
ROWS_PER_W = B // NW
ROW_W = T * V
TCH = 512
NCH = T // TCH
CW = TCH * V
GROUPS = TCH // 16

_mesh = plsc.VectorSubcoreMesh(
    core_axis_name="c", subcore_axis_name="s",
    num_cores=NUM_CORES, num_subcores=NUM_SUBCORES,
)


def _argmax_tree(buf, idxvs):
    xs = [
        plsc.load_gather(buf.at[pl.ds(8 * (v // 8), CW - 8 * (v // 8))],
                         [idxvs[v % 8]])
        for v in range(V)
    ]
    level = []
    for i in range(V // 2):
        a, b = xs[2 * i], xs[2 * i + 1]
        gt = b > a
        level.append((jnp.where(gt, b, a),
                      jnp.where(gt, jnp.int32(2 * i + 1), jnp.int32(2 * i))))
    level.append((xs[V - 1], jnp.full((16,), V - 1, jnp.int32)))
    while len(level) > 1:
        nxt = []
        for i in range(len(level) // 2):
            va, ia = level[2 * i]
            vb, ib = level[2 * i + 1]
            gt = vb > va
            nxt.append((jnp.where(gt, vb, va), jnp.where(gt, ib, ia)))
        if len(level) % 2:
            nxt.append(level[-1])
        level = nxt
    return level[0]


@functools.partial(
    pl.kernel,
    out_type=[
        jax.ShapeDtypeStruct((B * T,), jnp.int32),
        jax.ShapeDtypeStruct((B * T,), jnp.int32),
        jax.ShapeDtypeStruct((B * T,), jnp.float32),
    ],
    mesh=_mesh,
    compiler_params=pltpu.CompilerParams(needs_layout_passes=False),
    scratch_types=[
        pltpu.VMEM((CW,), jnp.float32),
        pltpu.VMEM((CW,), jnp.float32),
        pltpu.VMEM((T,), jnp.int32),
        pltpu.VMEM((T,), jnp.int32),
        pltpu.VMEM((T,), jnp.float32),
        pltpu.SemaphoreType.DMA,
        pltpu.SemaphoreType.DMA,
    ],
)
def _ctc_sc(lp_hbm, idx_hbm, val_hbm, mp_hbm,
            buf0, buf1, idxrow, valrow, mprow, sem0, sem1):
    wid = lax.axis_index("s") * NUM_CORES + lax.axis_index("c")
    iota16 = lax.iota(jnp.int32, 16)
    iotaVr = [iota16 * V + r for r in range(8)]
    bufs = (buf0, buf1)
    sems = (sem0, sem1)

    def src(step):
        r, c = divmod(step, NCH)
        b = wid * ROWS_PER_W + r
        off = pl.multiple_of(b * ROW_W + c * CW, 8)
        return lp_hbm.at[pl.ds(off, CW)]

    handles = {0: pltpu.async_copy(src(0), bufs[0], sems[0])}
    for step in range(ROWS_PER_W * NCH):
        r, c = divmod(step, NCH)
        buf = bufs[step % 2]
        handles.pop(step).wait()
        if step + 1 < ROWS_PER_W * NCH:
            handles[step + 1] = pltpu.async_copy(
                src(step + 1), bufs[(step + 1) % 2], sems[(step + 1) % 2])

        t_off = c * TCH

        @plsc.parallel_loop(0, GROUPS, unroll=2)
        def _pass1(g):
            gbase = g * (16 * V)
            cmax, cidx = _argmax_tree(buf, [gbase + iv for iv in iotaVr])
            start = t_off + g * 16
            idxrow[pl.ds(start, 16)] = cidx
            mprow[pl.ds(start, 16)] = jnp.exp(cmax)

        @plsc.parallel_loop(0, GROUPS, unroll=2)
        def _pass2(g):
            start = t_off + g * 16
            cur = idxrow[pl.ds(start, 16)]
            tv = start + iota16
            prev = plsc.load_gather(idxrow, [jnp.maximum(tv - 1, 0)])
            valid = (cur != 0) & ((cur != prev) | (tv == 0))
            valrow[pl.ds(start, 16)] = valid.astype(jnp.int32)

        if c == NCH - 1:
            off = pl.multiple_of((wid * ROWS_PER_W + r) * T, 8)
            pltpu.sync_copy(idxrow, idx_hbm.at[pl.ds(off, T)])
            pltpu.sync_copy(valrow, val_hbm.at[pl.ds(off, T)])
            pltpu.sync_copy(mprow, mp_hbm.at[pl.ds(off, T)])


def kernel(log_probs):
    lp = log_probs.reshape(-1)
    idx, val, mp = _ctc_sc(lp)
    return (
        idx.reshape(B, T),
        val.reshape(B, T).astype(bool),
        mp.reshape(B, T),
    )


# speedup vs baseline: 3.6341x; 3.4858x over previous
import jax
import jax.numpy as jnp
from jax import lax
from jax.experimental import pallas as pl
from jax.experimental.pallas import tpu as pltpu
from jax.experimental.pallas import tpu_sc as plsc

B = 128
T = 2048
V = 29
NUM_CORES = 2
NUM_SUBCORES = 16
NW = NUM_CORES * NUM_SUBCORES
ROWS_PER_W = B // NW
NBT = B // 8
NTT = T // 128
TTC = NTT // 2

_mesh = plsc.VectorSubcoreMesh(
    core_axis_name="c", subcore_axis_name="s",
    num_cores=NUM_CORES, num_subcores=NUM_SUBCORES,
)


def _argmax_tree(xs):
    level = []
    for i in range(V // 2):
        a, b = xs[2 * i], xs[2 * i + 1]
        gt = b > a
        level.append((jnp.where(gt, b, a),
                      jnp.where(gt, jnp.int32(2 * i + 1), jnp.int32(2 * i))))
    level.append((xs[V - 1], jnp.full((16,), V - 1, jnp.int32)))
    while len(level) > 1:
        nxt = []
        for i in range(len(level) // 2):
            va, ia = level[2 * i]
            vb, ib = level[2 * i + 1]
            gt = vb > va
            nxt.append((jnp.where(gt, vb, va), jnp.where(gt, ib, ia)))
        if len(level) % 2:
            nxt.append(level[-1])
        level = nxt
    return level[0]


@jax.jit
def _ctc_sc(lp5):
    @pl.kernel(
        out_type=[
            jax.ShapeDtypeStruct((NBT, NTT, 8, 128), jnp.int32),
            jax.ShapeDtypeStruct((NBT, NTT, 8, 128), jnp.int32),
            jax.ShapeDtypeStruct((NBT, NTT, 8, 128), jnp.float32),
        ],
        mesh=_mesh,
        compiler_params=pltpu.CompilerParams(needs_layout_passes=False),
        scratch_types=[
            pltpu.VMEM((V, TTC, 128), jnp.float32),
            pltpu.VMEM((V, TTC, 128), jnp.float32),
            pltpu.VMEM((NTT, 128), jnp.int32),
            pltpu.VMEM((NTT, 128), jnp.int32),
            pltpu.VMEM((NTT, 128), jnp.float32),
            pltpu.SemaphoreType.DMA,
            pltpu.SemaphoreType.DMA,
        ],
    )
    def k(lp_hbm, idx_hbm, val_hbm, mp_hbm,
          buf0, buf1, idxrow, valrow, mprow, sem0, sem1):
        wid = lax.axis_index("s") * NUM_CORES + lax.axis_index("c")
        iota16 = lax.iota(jnp.int32, 16)
        bufs = (buf0, buf1)
        sems = (sem0, sem1)
        nsteps = ROWS_PER_W * 2

        def src(step):
            r, half = divmod(step, 2)
            b = wid * ROWS_PER_W + r
            return lp_hbm.at[:, b // 8, pl.ds(half * TTC, TTC), b % 8, :]

        handles = {0: pltpu.async_copy(src(0), bufs[0], sems[0])}
        for step in range(nsteps):
            r, half = divmod(step, 2)
            buf = bufs[step % 2]
            handles.pop(step).wait()
            if step + 1 < nsteps:
                handles[step + 1] = pltpu.async_copy(
                    src(step + 1), bufs[(step + 1) % 2], sems[(step + 1) % 2])

            @plsc.parallel_loop(0, TTC * 8, unroll=2)
            def _pass1(g):
                tt_l = g // 8
                l0 = (g % 8) * 16
                xs = [buf[v, tt_l, pl.ds(l0, 16)] for v in range(V)]
                cmax, cidx = _argmax_tree(xs)
                idxrow[half * TTC + tt_l, pl.ds(l0, 16)] = cidx
                mprow[half * TTC + tt_l, pl.ds(l0, 16)] = jnp.exp(cmax)

            @plsc.parallel_loop(0, TTC * 8, unroll=2)
            def _pass2(g):
                tt_l = half * TTC + g // 8
                l0 = (g % 8) * 16
                cur = idxrow[tt_l, pl.ds(l0, 16)]
                t = tt_l * 128 + l0 + iota16
                pt = jnp.maximum(t - 1, 0)
                prev = plsc.load_gather(
                    idxrow, [lax.shift_right_logical(pt, 7), pt & 127])
                valid = (cur != 0) & ((cur != prev) | (t == 0))
                valrow[tt_l, pl.ds(l0, 16)] = valid.astype(jnp.int32)

            if half == 1:
                b = wid * ROWS_PER_W + r
                pltpu.sync_copy(idxrow, idx_hbm.at[b // 8, :, b % 8, :])
                pltpu.sync_copy(valrow, val_hbm.at[b // 8, :, b % 8, :])
                pltpu.sync_copy(mprow, mp_hbm.at[b // 8, :, b % 8, :])

    return k(lp5)


def kernel(log_probs):
    lp5 = log_probs.reshape(NBT, 8, NTT, 128, V).transpose(4, 0, 2, 1, 3)
    idx4, val4, mp4 = _ctc_sc(lp5)

    def unview(x4):
        return x4.transpose(0, 2, 1, 3).reshape(B, T)

    return (
        unview(idx4),
        unview(val4).astype(bool),
        unview(mp4),
    )
